# baseline (speedup 1.0000x reference)
import jax
import jax.numpy as jnp
from jax.experimental import pallas as pl


_GRID = 10
_X_ROWS = 10000 // _GRID
_EA_ROWS = 40000 // _GRID


def _copy_body(x_ref, ea_ref, x_out_ref, ea_out_ref):
    x_out_ref[...] = x_ref[...]
    ea_out_ref[...] = ea_ref[...]


def kernel(x, edge_index, edge_attr):
    del edge_index
    ea2 = edge_attr.reshape(40000, 128)
    x_out, ea_out = pl.pallas_call(
        _copy_body,
        grid=(_GRID,),
        in_specs=[
            pl.BlockSpec((_X_ROWS, 128), lambda i: (i, 0)),
            pl.BlockSpec((_EA_ROWS, 128), lambda i: (i, 0)),
        ],
        out_specs=[
            pl.BlockSpec((_X_ROWS, 128), lambda i: (i, 0)),
            pl.BlockSpec((_EA_ROWS, 128), lambda i: (i, 0)),
        ],
        out_shape=[
            jax.ShapeDtypeStruct((10000, 128), jnp.float32),
            jax.ShapeDtypeStruct((40000, 128), jnp.float32),
        ],
    )(x, ea2)
    return (x_out, ea_out.reshape(320000, 16))
